# baseline (speedup 1.0000x reference)
import jax
import jax.numpy as jnp
from jax.experimental import pallas as pl


def _gcn_block_kernel(a_ref, b_ref, o_ref):
    a = a_ref[...].astype(jnp.bfloat16)
    acc = jnp.dot(a, b_ref[...], preferred_element_type=jnp.float32)
    o_ref[...] = jnp.where(acc >= 0, acc, 0.5 * acc)


def kernel(adj, embeds):
    n, k = adj.shape
    d = embeds.shape[1]
    bm = 400
    emb_bf = embeds.astype(jnp.bfloat16)
    return pl.pallas_call(
        _gcn_block_kernel,
        grid=(n // bm,),
        in_specs=[
            pl.BlockSpec((bm, k), lambda i: (i, 0)),
            pl.BlockSpec((k, d), lambda i: (0, 0)),
        ],
        out_specs=pl.BlockSpec((bm, d), lambda i: (i, 0)),
        out_shape=jax.ShapeDtypeStruct((n, d), jnp.float32),
    )(adj, emb_bf)
